# baseline (speedup 1.0000x reference)
import jax
import jax.numpy as jnp
from jax.experimental import pallas as pl
from jax.experimental.pallas import tpu as pltpu

_B, _S, _D = 4, 8192, 2048
_ST, _EN = 0, 1024
_W = _EN - _ST
_R = _B * _S

_SIZES = [128, 384, 512, 1024] + [2048] * 14 + [1024, 512, 384, 128]
_STARTS = []
_off = 0
for _c in _SIZES:
    _STARTS.append(_off)
    _off += _c
assert _off == _R
_N = len(_SIZES)
_K = 3
_CMAX = 2048


def _body(eot_ref, w_hbm_ref, hid_ref, out_ref,
          bufs, w_s, rows_s, new_s, in_sems, out_sems, gsem, wsem):
    def fill(i):
        k = i % _K
        cp = pltpu.make_async_copy(
            hid_ref.at[pl.ds(_STARTS[i], _SIZES[i]), :],
            bufs.at[pl.ds(k * _CMAX, _SIZES[i]), :],
            in_sems.at[k],
        )
        cp.start()
        return cp

    def drain(i):
        k = i % _K
        cp = pltpu.make_async_copy(
            bufs.at[pl.ds(k * _CMAX, _SIZES[i]), :],
            out_ref.at[pl.ds(_STARTS[i], _SIZES[i]), :],
            out_sems.at[k],
        )
        cp.start()
        return cp

    fills = [None] * _N
    drains = [None] * _N
    for i in range(min(_K, _N)):
        fills[i] = fill(i)

    wcp = pltpu.make_async_copy(w_hbm_ref, w_s, wsem)
    wcp.start()
    gathers = []
    for b in range(_B):
        r = b * _S + eot_ref[b]
        cp = pltpu.make_async_copy(
            hid_ref.at[pl.ds(r, 1), pl.ds(_ST, _W)],
            rows_s.at[pl.ds(b, 1)],
            gsem,
        )
        cp.start()
        gathers.append(cp)
    for cp in gathers:
        cp.wait()
    wcp.wait()
    t = rows_s[...]
    rot = jax.lax.dot_general(
        t, w_s[...], (((1,), (0,)), ((), ())),
        preferred_element_type=jnp.float32,
    )
    new_s[...] = jax.lax.dot_general(
        rot, w_s[...], (((1,), (1,)), ((), ())),
        preferred_element_type=jnp.float32,
    )

    for i in range(_N):
        fills[i].wait()
        drains[i] = drain(i)
        p = i - 1
        if p >= 0 and p + _K < _N:
            drains[p].wait()
            fills[p + _K] = fill(p + _K)
    for i in range(max(0, _N - _K), _N):
        drains[i].wait()

    patches = []
    for b in range(_B):
        r = b * _S + eot_ref[b]
        cp = pltpu.make_async_copy(
            new_s.at[pl.ds(b, 1)],
            out_ref.at[pl.ds(r, 1), pl.ds(_ST, _W)],
            gsem,
        )
        cp.start()
        patches.append(cp)
    for cp in patches:
        cp.wait()


def kernel(hidden_states, eot_indices, W):
    eot = eot_indices.astype(jnp.int32)
    hid2 = hidden_states.reshape(_R, _D)
    out = pl.pallas_call(
        _body,
        in_specs=[
            pl.BlockSpec(memory_space=pltpu.MemorySpace.SMEM),
            pl.BlockSpec(memory_space=pltpu.MemorySpace.HBM),
            pl.BlockSpec(memory_space=pltpu.MemorySpace.HBM),
        ],
        out_specs=pl.BlockSpec(memory_space=pltpu.MemorySpace.HBM),
        out_shape=jax.ShapeDtypeStruct((_R, _D), jnp.float32),
        scratch_shapes=[
            pltpu.VMEM((_K * _CMAX, _D), jnp.float32),
            pltpu.VMEM((_W, _W), jnp.float32),
            pltpu.VMEM((_B, _W), jnp.float32),
            pltpu.VMEM((_B, _W), jnp.float32),
            pltpu.SemaphoreType.DMA((_K,)),
            pltpu.SemaphoreType.DMA((_K,)),
            pltpu.SemaphoreType.DMA,
            pltpu.SemaphoreType.DMA,
        ],
    )(eot, W, hid2)
    return out.reshape(_B, _S, _D)


# speedup vs baseline: 1.0125x; 1.0125x over previous
import jax
import jax.numpy as jnp
from jax.experimental import pallas as pl
from jax.experimental.pallas import tpu as pltpu

_B, _S, _D = 4, 8192, 2048
_ST, _EN = 0, 1024
_W = _EN - _ST
_BS = 1024
_NB = _S // _BS


def _gather_cps(eot_ref, hid_any_ref, rows_s, sem):
    cps = []
    for bb in range(_B):
        e = eot_ref[bb]
        cps.append(pltpu.make_async_copy(
            hid_any_ref.at[pl.ds(bb, 1), pl.ds(e, 1), pl.ds(_ST, _W)],
            rows_s.at[pl.ds(bb, 1)],
            sem,
        ))
    return cps


def _body(eot_ref, w_hbm_ref, hid_blk_ref, hid_any_ref, out_ref,
          w_s, rows_s, new_s, sem, wsem):
    b = pl.program_id(0)
    j = pl.program_id(1)

    out_ref[...] = hid_blk_ref[...]

    @pl.when((b == 0) & (j == 0))
    def _start_dmas():
        pltpu.make_async_copy(w_hbm_ref, w_s, wsem).start()
        for cp in _gather_cps(eot_ref, hid_any_ref, rows_s, sem):
            cp.start()

    @pl.when((b == 0) & (j == 1))
    def _rotate():
        for cp in _gather_cps(eot_ref, hid_any_ref, rows_s, sem):
            cp.wait()
        pltpu.make_async_copy(w_hbm_ref, w_s, wsem).wait()
        t = rows_s[...].reshape(_B, _W)
        r = jax.lax.dot_general(
            t, w_s[...], (((1,), (0,)), ((), ())),
            preferred_element_type=jnp.float32,
        )
        inv = jax.lax.dot_general(
            r, w_s[...], (((1,), (1,)), ((), ())),
            preferred_element_type=jnp.float32,
        )
        new_s[...] = inv.reshape(_B, 1, _W)

    @pl.when(j == _NB - 1)
    def _patch():
        local = eot_ref[b] % _BS
        out_ref[pl.ds(0, 1), pl.ds(local, 1), pl.ds(_ST, _W)] = (
            new_s[pl.ds(b, 1)]
        )


def _reorder(b, j, eot_ref):
    k_e = eot_ref[b] // _BS
    jj = jnp.where(j < k_e, j, jnp.where(j < _NB - 1, j + 1, k_e))
    return (b, jj, 0)


def kernel(hidden_states, eot_indices, W):
    eot = eot_indices.astype(jnp.int32)
    grid_spec = pltpu.PrefetchScalarGridSpec(
        num_scalar_prefetch=1,
        grid=(_B, _NB),
        in_specs=[
            pl.BlockSpec(memory_space=pltpu.MemorySpace.HBM),
            pl.BlockSpec((1, _BS, _D), _reorder),
            pl.BlockSpec(memory_space=pltpu.MemorySpace.HBM),
        ],
        out_specs=pl.BlockSpec((1, _BS, _D), _reorder),
        scratch_shapes=[
            pltpu.VMEM((_W, _W), jnp.float32),
            pltpu.VMEM((_B, 1, _W), jnp.float32),
            pltpu.VMEM((_B, 1, _W), jnp.float32),
            pltpu.SemaphoreType.DMA,
            pltpu.SemaphoreType.DMA,
        ],
    )
    return pl.pallas_call(
        _body,
        grid_spec=grid_spec,
        out_shape=jax.ShapeDtypeStruct((_B, _S, _D), jnp.float32),
        compiler_params=pltpu.CompilerParams(
            dimension_semantics=("arbitrary", "arbitrary"),
        ),
    )(eot, W, hidden_states, hidden_states)
